# Initial kernel scaffold; baseline (speedup 1.0000x reference)
import jax
import jax.numpy as jnp
from jax.experimental import pallas as pl


def kernel(embs, SPE, path_emb_weight):
    raise NotImplementedError("write your pallas kernel here")


# speedup vs baseline: 8.2461x; 8.2461x over previous
import functools
import math

import jax
import jax.numpy as jnp
from jax import lax
from jax.experimental import pallas as pl
from jax.experimental.pallas import tpu as pltpu
from jax.experimental.pallas import tpu_sc as plsc

N = 10000
E = 320000
D = 128
NUM_PATHS = 16382
P = 16384

NC = 2
NS = 16
L = 16
NW = NC * NS
EPW = E // NW
C = 80
NCHUNK = EPW // C
G = C // L
NP = 10240
RPT = NP // NS

INV_SQRT_D = 1.0 / math.sqrt(float(D))

_mesh = plsc.VectorSubcoreMesh(core_axis_name="c", subcore_axis_name="s")
_sc_params = pltpu.CompilerParams(needs_layout_passes=False)


def _ln_body(e_ref, x_ref):
    e = e_ref[...]
    mu = jnp.mean(e, axis=-1, keepdims=True)
    var = jnp.mean((e - mu) ** 2, axis=-1, keepdims=True)
    x_ref[...] = (e - mu) / jnp.sqrt(var + 1e-5)


def _layer_norm(embs):
    return pl.pallas_call(
        _ln_body,
        out_shape=jax.ShapeDtypeStruct((N, D), jnp.float32),
    )(embs)


@functools.partial(
    pl.kernel,
    out_type=(
        jax.ShapeDtypeStruct((E,), jnp.float32),
        jax.ShapeDtypeStruct((NW * N,), jnp.float32),
        jax.ShapeDtypeStruct((NW * N,), jnp.float32),
    ),
    mesh=_mesh,
    compiler_params=_sc_params,
    scratch_types=[
        pltpu.VMEM((C,), jnp.int32),
        pltpu.VMEM((C,), jnp.int32),
        pltpu.VMEM((C,), jnp.int32),
        pltpu.VMEM((C, D), jnp.float32),
        pltpu.VMEM((C, D), jnp.float32),
        pltpu.VMEM((C,), jnp.float32),
        pltpu.VMEM((P,), jnp.float32),
        pltpu.VMEM((N,), jnp.float32),
        pltpu.VMEM((N,), jnp.float32),
        pltpu.SemaphoreType.DMA,
        pltpu.SemaphoreType.DMA,
    ],
)
def _pass1(x_hbm, src_hbm, dst_hbm, pid_hbm, path_hbm,
           w_hbm, s_hbm, sp_hbm,
           src_v, dst_v, pid_v, rs_v, rd_v, w_v, path_v, s_v, sp_v,
           sem1, sem2):
    cid = lax.axis_index("c")
    sid = lax.axis_index("s")
    wid = sid * NC + cid
    ebase = wid * EPW

    pltpu.sync_copy(path_hbm, path_v)

    def zero_body(i, _):
        s_v[pl.ds(i * L, L)] = jnp.zeros((L,), jnp.float32)
        sp_v[pl.ds(i * L, L)] = jnp.zeros((L,), jnp.float32)
        return 0

    lax.fori_loop(0, N // L, zero_body, 0, unroll=4)

    lane = lax.iota(jnp.int32, L)

    def chunk_body(c, _):
        base = ebase + c * C
        pltpu.sync_copy(src_hbm.at[pl.ds(base, C)], src_v)
        pltpu.sync_copy(dst_hbm.at[pl.ds(base, C)], dst_v)
        pltpu.sync_copy(pid_hbm.at[pl.ds(base, C)], pid_v)
        cp1 = pltpu.async_copy(x_hbm.at[src_v], rs_v, sem1)
        cp2 = pltpu.async_copy(x_hbm.at[dst_v], rd_v, sem2)
        cp1.wait()
        cp2.wait()
        for g in range(G):
            vec = jnp.zeros((L,), jnp.float32)
            for j in range(L):
                e = g * L + j
                acc = rs_v[e, pl.ds(0, L)] * rd_v[e, pl.ds(0, L)]
                for k in range(1, D // L):
                    acc = acc + rs_v[e, pl.ds(k * L, L)] * rd_v[e, pl.ds(k * L, L)]
                vec = jnp.where(lane == j, jnp.sum(acc), vec)
            w16 = jnp.minimum(jnp.exp(vec * INV_SQRT_D), 5.0)
            w_v[pl.ds(g * L, L)] = w16
            src16 = src_v[pl.ds(g * L, L)]
            plsc.addupdate_scatter(s_v, [src16], w16)
            pid16 = pid_v[pl.ds(g * L, L)]
            p16 = plsc.load_gather(path_v, [pid16])
            wp16 = jnp.minimum(jnp.exp(p16), 5.0)
            plsc.addupdate_scatter(sp_v, [src16], wp16)
        pltpu.sync_copy(w_v, w_hbm.at[pl.ds(base, C)])
        return 0

    lax.fori_loop(0, NCHUNK, chunk_body, 0)

    pltpu.sync_copy(s_v, s_hbm.at[pl.ds(wid * N, N)])
    pltpu.sync_copy(sp_v, sp_hbm.at[pl.ds(wid * N, N)])


def _combine_body(sp_ref, spp_ref, s_ref, sp_out_ref):
    s_ref[...] = jnp.sum(sp_ref[...], axis=0)
    sp_out_ref[...] = jnp.sum(spp_ref[...], axis=0)


def _combine_denoms(s_parts, sp_parts):
    return pl.pallas_call(
        _combine_body,
        out_shape=(
            jax.ShapeDtypeStruct((N,), jnp.float32),
            jax.ShapeDtypeStruct((N,), jnp.float32),
        ),
    )(s_parts, sp_parts)


@functools.partial(
    pl.kernel,
    out_type=jax.ShapeDtypeStruct((NC, NP, D), jnp.float32),
    mesh=_mesh,
    compiler_params=_sc_params,
    scratch_types=[
        pltpu.VMEM((C,), jnp.int32),
        pltpu.VMEM((C,), jnp.int32),
        pltpu.VMEM((C,), jnp.int32),
        pltpu.VMEM((C, D), jnp.float32),
        pltpu.VMEM((C,), jnp.float32),
        pltpu.VMEM((N,), jnp.float32),
        pltpu.VMEM((N,), jnp.float32),
        pltpu.VMEM((P,), jnp.float32),
        pltpu.VMEM_SHARED((NP, D), jnp.float32),
        pltpu.SemaphoreType.DMA,
    ],
)
def _pass2(x_hbm, src_hbm, dst_hbm, pid_hbm, path_hbm, w_hbm,
           s_hbm, sp_hbm, zeros_hbm,
           out_hbm,
           src_v, dst_v, pid_v, rd_v, w_v, s_v, sp_v, path_v,
           acc_sh, sem1):
    cid = lax.axis_index("c")
    sid = lax.axis_index("s")
    wid = sid * NC + cid
    ebase = wid * EPW

    pltpu.sync_copy(path_hbm, path_v)
    pltpu.sync_copy(s_hbm, s_v)
    pltpu.sync_copy(sp_hbm, sp_v)
    pltpu.sync_copy(zeros_hbm.at[pl.ds(sid * RPT, RPT)],
                    acc_sh.at[pl.ds(sid * RPT, RPT)])
    plsc.subcore_barrier()

    def chunk_body(c, _):
        base = ebase + c * C
        pltpu.sync_copy(src_hbm.at[pl.ds(base, C)], src_v)
        pltpu.sync_copy(dst_hbm.at[pl.ds(base, C)], dst_v)
        pltpu.sync_copy(pid_hbm.at[pl.ds(base, C)], pid_v)
        pltpu.sync_copy(w_hbm.at[pl.ds(base, C)], w_v)
        pltpu.async_copy(x_hbm.at[dst_v], rd_v, sem1).wait()
        for g in range(G):
            w16 = w_v[pl.ds(g * L, L)]
            src16 = src_v[pl.ds(g * L, L)]
            pid16 = pid_v[pl.ds(g * L, L)]
            s16 = plsc.load_gather(s_v, [src16])
            sp16 = plsc.load_gather(sp_v, [src16])
            p16 = plsc.load_gather(path_v, [pid16])
            wp16 = jnp.minimum(jnp.exp(p16), 5.0)
            c16 = w16 / s16 + wp16 / sp16
            for j in range(L):
                e = g * L + j
                splat = jnp.broadcast_to(c16[j], (L,))
                for k in range(D // L):
                    rd_v[e, pl.ds(k * L, L)] = rd_v[e, pl.ds(k * L, L)] * splat
        pltpu.sync_copy(rd_v, acc_sh.at[src_v], add=True)
        return 0

    lax.fori_loop(0, NCHUNK, chunk_body, 0)

    plsc.subcore_barrier()
    pltpu.sync_copy(acc_sh.at[pl.ds(sid * RPT, RPT)],
                    out_hbm.at[cid, pl.ds(sid * RPT, RPT)])


def _final_body(p_ref, out_ref):
    out_ref[...] = p_ref[0] + p_ref[1]


def _final_combine(parts):
    return pl.pallas_call(
        _final_body,
        out_shape=jax.ShapeDtypeStruct((N, D), jnp.float32),
    )(parts)


def kernel(embs, SPE, path_emb_weight):
    src = SPE[:, 0]
    dst = SPE[:, 1]
    pid = SPE[:, 2]
    path = jnp.pad(path_emb_weight.reshape(-1), (0, P - NUM_PATHS))
    zeros = jnp.zeros((NP, D), jnp.float32)
    x = _layer_norm(embs)
    w, s_parts, sp_parts = _pass1(x, src, dst, pid, path)
    s, sp = _combine_denoms(s_parts.reshape(NW, N), sp_parts.reshape(NW, N))
    parts = _pass2(x, src, dst, pid, path, w, s, sp, zeros)
    return _final_combine(parts[:, :N, :])


# speedup vs baseline: 14.5092x; 1.7595x over previous
import functools
import math

import jax
import jax.numpy as jnp
from jax import lax
from jax.experimental import pallas as pl
from jax.experimental.pallas import tpu as pltpu
from jax.experimental.pallas import tpu_sc as plsc

N = 10000
E = 320000
D = 128
NUM_PATHS = 16382
P = 16384

NC = 2
NS = 16
L = 16
NW = NC * NS
EPW = E // NW
C = 80
NCHUNK = EPW // C
G = C // L
NP = 10240
RPT = NP // NS
SPW = EPW * 3

INV_SQRT_D = 1.0 / math.sqrt(float(D))

_mesh = plsc.VectorSubcoreMesh(core_axis_name="c", subcore_axis_name="s")
_sc_params = pltpu.CompilerParams(needs_layout_passes=False)
_sc_params_sct = pltpu.CompilerParams(needs_layout_passes=False,
                                      use_tc_tiling_on_sc=False)


def _ln_body(e_ref, x_ref, xb_ref):
    e = e_ref[...]
    mu = jnp.mean(e, axis=-1, keepdims=True)
    var = jnp.mean((e - mu) ** 2, axis=-1, keepdims=True)
    x = (e - mu) / jnp.sqrt(var + 1e-5)
    x_ref[...] = x
    xb_ref[...] = x.astype(jnp.bfloat16)


def _layer_norm(embs):
    return pl.pallas_call(
        _ln_body,
        out_shape=(
            jax.ShapeDtypeStruct((N, D), jnp.float32),
            jax.ShapeDtypeStruct((N, D), jnp.bfloat16),
        ),
    )(embs)


@functools.partial(
    pl.kernel,
    out_type=(
        jax.ShapeDtypeStruct((E,), jnp.int32),
        jax.ShapeDtypeStruct((NW * N,), jnp.float32),
        jax.ShapeDtypeStruct((NW * N,), jnp.float32),
    ),
    mesh=_mesh,
    compiler_params=_sc_params_sct,
    scratch_types=[
        pltpu.VMEM((SPW,), jnp.int32),
        pltpu.VMEM((C,), jnp.int32),
        pltpu.VMEM((C,), jnp.int32),
        pltpu.VMEM((C,), jnp.int32),
        pltpu.VMEM((C,), jnp.int32),
        pltpu.VMEM((C, D // 2), jnp.int32),
        pltpu.VMEM((C, D // 2), jnp.int32),
        pltpu.VMEM((C, D // 2), jnp.int32),
        pltpu.VMEM((C, D // 2), jnp.int32),
        pltpu.VMEM((EPW,), jnp.int32),
        pltpu.VMEM((P,), jnp.float32),
        pltpu.VMEM((N,), jnp.float32),
        pltpu.VMEM((N,), jnp.float32),
        pltpu.SemaphoreType.DMA,
        pltpu.SemaphoreType.DMA,
        pltpu.SemaphoreType.DMA,
        pltpu.SemaphoreType.DMA,
    ],
)
def _pass1(xb_hbm, spe_hbm, path_hbm,
           w_hbm, s_hbm, sp_hbm,
           spe_v, si0, si1, di0, di1, rs0, rs1, rd0, rd1,
           w_v, path_v, s_v, sp_v,
           sem_s0, sem_s1, sem_d0, sem_d1):
    cid = lax.axis_index("c")
    sid = lax.axis_index("s")
    wid = sid * NC + cid
    ebase = wid * EPW

    pltpu.sync_copy(spe_hbm.at[pl.ds(wid * SPW, SPW)], spe_v)
    pltpu.sync_copy(path_hbm, path_v)

    def zero_body(i, _):
        s_v[pl.ds(i * L, L)] = jnp.zeros((L,), jnp.float32)
        sp_v[pl.ds(i * L, L)] = jnp.zeros((L,), jnp.float32)
        return 0

    lax.fori_loop(0, N // L, zero_body, 0, unroll=4)

    lane3 = lax.iota(jnp.int32, L) * 3

    def prep(c, si, di):
        for g in range(G):
            eoff = jnp.full((L,), 0, jnp.int32) + (c * C + g * L) * 3 + lane3
            si[pl.ds(g * L, L)] = plsc.load_gather(spe_v, [eoff])
            di[pl.ds(g * L, L)] = plsc.load_gather(spe_v, [eoff + 1])

    def issue(si, di, rs, rd, sem_s, sem_d):
        pltpu.async_copy(xb_hbm.at[si], rs, sem_s)
        pltpu.async_copy(xb_hbm.at[di], rd, sem_d)

    def wait(si, di, rs, rd, sem_s, sem_d):
        pltpu.make_async_copy(xb_hbm.at[si], rs, sem_s).wait()
        pltpu.make_async_copy(xb_hbm.at[di], rd, sem_d).wait()

    lane = lax.iota(jnp.int32, L)

    def compute(c, si, rs, rd):
        for g in range(G):
            vec = jnp.zeros((L,), jnp.float32)
            for j in range(L):
                e = g * L + j
                acc = jnp.zeros((L,), jnp.float32)
                for k in range(D // (2 * L)):
                    sa, sb = plsc.unpack(
                        plsc.bitcast(rs[e, pl.ds(k * L, L)], jnp.bfloat16),
                        format=plsc.PackFormat.INTERLEAVED)
                    da, db = plsc.unpack(
                        plsc.bitcast(rd[e, pl.ds(k * L, L)], jnp.bfloat16),
                        format=plsc.PackFormat.INTERLEAVED)
                    acc = acc + sa * da + sb * db
                vec = jnp.where(lane == j, jnp.sum(acc), vec)
            w16 = jnp.minimum(jnp.exp(vec * INV_SQRT_D), 5.0)
            src16 = si[pl.ds(g * L, L)]
            eoff = jnp.full((L,), 0, jnp.int32) + (c * C + g * L) * 3 + lane3
            pid16 = plsc.load_gather(spe_v, [eoff + 2])
            plsc.addupdate_scatter(s_v, [src16], w16)
            p16 = plsc.load_gather(path_v, [pid16])
            wp16 = jnp.minimum(jnp.exp(p16), 5.0)
            plsc.addupdate_scatter(sp_v, [src16], wp16)
            packed = plsc.pack(w16, wp16, format=plsc.PackFormat.INTERLEAVED)
            w_v[pl.ds(c * C + g * L, L)] = plsc.bitcast(packed, jnp.int32)

    prep(0, si0, di0)
    issue(si0, di0, rs0, rd0, sem_s0, sem_d0)

    def macro_body(m, _):
        c0 = m * 2
        prep(c0 + 1, si1, di1)
        issue(si1, di1, rs1, rd1, sem_s1, sem_d1)
        wait(si0, di0, rs0, rd0, sem_s0, sem_d0)
        compute(c0, si0, rs0, rd0)
        prep(c0 + 2, si0, di0)
        issue(si0, di0, rs0, rd0, sem_s0, sem_d0)
        wait(si1, di1, rs1, rd1, sem_s1, sem_d1)
        compute(c0 + 1, si1, rs1, rd1)
        return 0

    lax.fori_loop(0, (NCHUNK - 1) // 2, macro_body, 0)

    wait(si0, di0, rs0, rd0, sem_s0, sem_d0)
    compute(NCHUNK - 1, si0, rs0, rd0)

    pltpu.sync_copy(w_v, w_hbm.at[pl.ds(ebase, EPW)])
    pltpu.sync_copy(s_v, s_hbm.at[pl.ds(wid * N, N)])
    pltpu.sync_copy(sp_v, sp_hbm.at[pl.ds(wid * N, N)])


def _combine_body(sp_ref, spp_ref, s_ref, sp_out_ref):
    s_ref[...] = jnp.sum(sp_ref[...], axis=0)
    sp_out_ref[...] = jnp.sum(spp_ref[...], axis=0)


def _combine_denoms(s_parts, sp_parts):
    return pl.pallas_call(
        _combine_body,
        out_shape=(
            jax.ShapeDtypeStruct((N,), jnp.float32),
            jax.ShapeDtypeStruct((N,), jnp.float32),
        ),
    )(s_parts, sp_parts)


H = D // 2
EPS = E // NS
NCHUNK2 = EPS // C
TRIO = C * 3


@functools.partial(
    pl.kernel,
    out_type=jax.ShapeDtypeStruct((NC, NP, H), jnp.float32),
    mesh=_mesh,
    compiler_params=_sc_params_sct,
    scratch_types=[
        pltpu.VMEM((TRIO,), jnp.int32),
        pltpu.VMEM((TRIO,), jnp.int32),
        pltpu.VMEM((C,), jnp.int32),
        pltpu.VMEM((C,), jnp.int32),
        pltpu.VMEM((C,), jnp.int32),
        pltpu.VMEM((C,), jnp.int32),
        pltpu.VMEM((C,), jnp.int32),
        pltpu.VMEM((C,), jnp.int32),
        pltpu.VMEM((C, H), jnp.float32),
        pltpu.VMEM((C, H), jnp.float32),
        pltpu.VMEM((N,), jnp.float32),
        pltpu.VMEM((N,), jnp.float32),
        pltpu.VMEM_SHARED((NP, H), jnp.float32),
        pltpu.SemaphoreType.DMA,
        pltpu.SemaphoreType.DMA,
        pltpu.SemaphoreType.DMA,
        pltpu.SemaphoreType.DMA,
        pltpu.SemaphoreType.DMA,
        pltpu.SemaphoreType.DMA,
        pltpu.SemaphoreType.DMA,
        pltpu.SemaphoreType.DMA,
    ],
)
def _pass2(xh_hbm, spe_hbm, wp_hbm, s_hbm, sp_hbm, zeros_hbm,
           out_hbm,
           t0, t1, p0, p1, si0, si1, di0, di1, rd0, rd1,
           s_v, sp_v, acc_sh,
           sem_t0, sem_t1, sem_p0, sem_p1, sem_g0, sem_g1, sem_c0, sem_c1):
    cid = lax.axis_index("c")
    sid = lax.axis_index("s")
    ebase = sid * EPS
    rowoff = cid * N

    pltpu.sync_copy(s_hbm, s_v)
    pltpu.sync_copy(sp_hbm, sp_v)
    pltpu.sync_copy(zeros_hbm.at[pl.ds(sid * RPT, RPT)],
                    acc_sh.at[pl.ds(sid * RPT, RPT)])
    plsc.subcore_barrier()

    lane3 = lax.iota(jnp.int32, L) * 3
    lane = lax.iota(jnp.int32, L)

    trio = (t0, t1)
    wpair = (p0, p1)
    slist = (si0, si1)
    dlist = (di0, di1)
    rows = (rd0, rd1)
    sem_t = (sem_t0, sem_t1)
    sem_p = (sem_p0, sem_p1)
    sem_g = (sem_g0, sem_g1)
    sem_c = (sem_c0, sem_c1)

    def issue_trio(b, c):
        pltpu.async_copy(spe_hbm.at[pl.ds((ebase + c * C) * 3, TRIO)],
                         trio[b], sem_t[b])

    def wait_trio(b, c):
        pltpu.make_async_copy(spe_hbm.at[pl.ds((ebase + c * C) * 3, TRIO)],
                              trio[b], sem_t[b]).wait()

    def issue_wpair(b, c):
        pltpu.async_copy(wp_hbm.at[pl.ds(ebase + c * C, C)],
                         wpair[b], sem_p[b])

    def wait_wpair(b, c):
        pltpu.make_async_copy(wp_hbm.at[pl.ds(ebase + c * C, C)],
                              wpair[b], sem_p[b]).wait()

    def prep(b):
        for g in range(G):
            eoff = jnp.full((L,), 0, jnp.int32) + g * L * 3 + lane3
            slist[b][pl.ds(g * L, L)] = plsc.load_gather(trio[b], [eoff])
            dlist[b][pl.ds(g * L, L)] = (
                plsc.load_gather(trio[b], [eoff + 1]) + rowoff)

    def issue_gather(b):
        pltpu.async_copy(xh_hbm.at[dlist[b]], rows[b], sem_g[b])

    def wait_gather(b):
        pltpu.make_async_copy(xh_hbm.at[dlist[b]], rows[b], sem_g[b]).wait()

    def issue_scatter(b):
        pltpu.async_copy(rows[b], acc_sh.at[slist[b]], sem_c[b], add=True)

    def wait_scatter(b):
        pltpu.make_async_copy(rows[b], acc_sh.at[slist[b]], sem_c[b]).wait()

    def scale(b):
        rd = rows[b]
        for g in range(G):
            pw = plsc.bitcast(wpair[b][pl.ds(g * L, L)], jnp.bfloat16)
            w16, wp16 = plsc.unpack(pw, format=plsc.PackFormat.INTERLEAVED)
            src16 = slist[b][pl.ds(g * L, L)]
            s16 = plsc.load_gather(s_v, [src16])
            sp16 = plsc.load_gather(sp_v, [src16])
            c16 = w16 / s16 + wp16 / sp16
            for j in range(L):
                e = g * L + j
                splat = jnp.broadcast_to(c16[j], (L,))
                for k in range(H // L):
                    rd[e, pl.ds(k * L, L)] = rd[e, pl.ds(k * L, L)] * splat

    def phase(c, b, prefetch_next, prefetch_next2):
        nb = 1 - b
        if prefetch_next:
            wait_trio(nb, c + 1)
            wait_scatter(nb)
            prep(nb)
            issue_gather(nb)
        if prefetch_next2:
            issue_trio(b, c + 2)
        wait_gather(b)
        wait_wpair(b, c)
        scale(b)
        issue_scatter(b)
        if prefetch_next2:
            issue_wpair(b, c + 2)

    issue_trio(0, 0)
    issue_wpair(0, 0)
    wait_trio(0, 0)
    prep(0)
    issue_gather(0)
    issue_trio(1, 1)
    issue_wpair(1, 1)
    for g in range(G):
        si1[pl.ds(g * L, L)] = jnp.full((L,), N + g * L, jnp.int32) + lane
    issue_scatter(1)

    def macro_body(m, _):
        c0 = m * 2
        phase(c0, 0, True, True)
        phase(c0 + 1, 1, True, True)
        return 0

    lax.fori_loop(0, (NCHUNK2 - 2) // 2, macro_body, 0)

    phase(NCHUNK2 - 2, 0, True, False)
    phase(NCHUNK2 - 1, 1, False, False)
    wait_scatter(0)
    wait_scatter(1)

    plsc.subcore_barrier()
    pltpu.sync_copy(acc_sh.at[pl.ds(sid * RPT, RPT)],
                    out_hbm.at[cid, pl.ds(sid * RPT, RPT)])


def _final_body(p_ref, out_ref):
    out_ref[...] = jnp.concatenate([p_ref[0], p_ref[1]], axis=-1)


def _final_combine(parts):
    return pl.pallas_call(
        _final_body,
        out_shape=jax.ShapeDtypeStruct((N, D), jnp.float32),
    )(parts)


def kernel(embs, SPE, path_emb_weight):
    spe_flat = SPE.reshape(E * 3)
    path = jnp.pad(path_emb_weight.reshape(-1), (0, P - NUM_PATHS))
    zeros = jnp.zeros((NP, H), jnp.float32)
    x, xb = _layer_norm(embs)
    xb32 = lax.bitcast_convert_type(xb.reshape(N, D // 2, 2), jnp.int32)
    xh = jnp.concatenate([x[:, :H], x[:, H:]], axis=0)
    wp_packed, s_parts, sp_parts = _pass1(xb32, spe_flat, path)
    s, sp = _combine_denoms(s_parts.reshape(NW, N), sp_parts.reshape(NW, N))
    parts = _pass2(xh, spe_flat, wp_packed, s, sp, zeros)
    return _final_combine(parts[:, :N, :])
